# baseline (speedup 1.0000x reference)
import functools
import math

import jax
import jax.numpy as jnp
import numpy as np
from jax import lax
from jax.experimental import pallas as pl
from jax.experimental.pallas import tpu as pltpu
from jax.experimental.pallas import tpu_sc as plsc

N = 10000
K = 16
IFZ = 128
AHZ = 12
AFZ = 16
QPZ = 4
VPZ = 8
DT = 832
DTU = 816

WL = math.sqrt(1.0 / 3.0)
WC = math.sqrt(2.0 / (9.0 * QPZ))

B1 = 400
B3 = 80


def _perm_rope():
    p = []
    for a in range(AHZ):
        p += [a * AFZ + i for i in range(0, AFZ, 2)]
        p += [a * AFZ + i for i in range(1, AFZ, 2)]
    return np.array(p)


def _perm_cmajor(npts):
    p = np.empty(3 * AHZ * npts, dtype=np.int64)
    for c in range(3):
        for a in range(AHZ):
            for q in range(npts):
                p[c * AHZ * npts + a * npts + q] = (a * npts + q) * 3 + c
    return p


def _perm_backw_cols():
    cols = np.arange(AHZ * AFZ + AHZ * (IFZ // 2) + AHZ * VPZ * 4)
    base = AHZ * AFZ + AHZ * (IFZ // 2)
    ipa = np.empty(3 * AHZ * VPZ, dtype=np.int64)
    for c in range(3):
        for av in range(AHZ * VPZ):
            ipa[c * AHZ * VPZ + av] = base + av * 3 + c
    cols[base:base + 3 * AHZ * VPZ] = ipa
    return cols


def _s1_body(x1_ref, x2f_ref, pe_ref, aff_ref, w_ref, biaw2_ref,
             table_ref, q1r_ref, tq_ref, bias_ref):
    x1 = x1_ref[:]
    proj = lax.dot_general(x1, w_ref[:], (((1,), (1,)), ((), ())),
                           preferred_element_type=jnp.float32)
    pe = pe_ref[:]
    cos = jnp.cos(pe)[:, None, :]
    sin = jnp.sin(pe)[:, None, :]
    b = x1.shape[0]

    def rope(q):
        q3 = q.reshape(b, AHZ, AFZ)
        v1 = q3[:, :, 0:8]
        v2 = q3[:, :, 8:16]
        r1 = v1 * cos - v2 * sin
        r2 = v1 * sin + v2 * cos
        return jnp.concatenate([r1, r2], axis=-1).reshape(b, AHZ * AFZ)

    aff = aff_ref[:]

    def affmul(v, s):
        outs = []
        for i in range(3):
            acc = aff[:, 4 * i + 3:4 * i + 4]
            for j in range(3):
                acc = acc + aff[:, 4 * i + j:4 * i + j + 1] * v[:, j * s:(j + 1) * s]
            outs.append(acc)
        return jnp.concatenate(outs, axis=-1)

    q1r_ref[:] = rope(proj[:, 0:192])
    k1r = rope(proj[:, 192:384])
    v1 = proj[:, 384:576]
    tq_ref[:] = affmul(proj[:, 576:720], 48)
    k2 = proj[:, 720:864]
    tv = affmul(proj[:, 864:1152], 96)
    pad = jnp.zeros((b, DT - DTU), jnp.float32)
    table_ref[:] = jnp.concatenate([k1r, v1, k2, tv, pad],
                                   axis=-1).astype(jnp.bfloat16)
    bias_ref[:] = lax.dot_general(x2f_ref[:], biaw2_ref[:],
                                  (((1,), (0,)), ((), ())),
                                  preferred_element_type=jnp.float32)


def _stage1(x1, x2f, pos_emb, affq, wcat, biaw2, *, interpret=False):
    grid = (N // B1,)
    return pl.pallas_call(
        _s1_body,
        grid=grid,
        in_specs=[
            pl.BlockSpec((B1, IFZ), lambda i: (i, 0)),
            pl.BlockSpec((B1, K * (IFZ // 2)), lambda i: (i, 0)),
            pl.BlockSpec((B1, AFZ // 2), lambda i: (i, 0)),
            pl.BlockSpec((B1, 16), lambda i: (i, 0)),
            pl.BlockSpec((1152, IFZ), lambda i: (0, 0)),
            pl.BlockSpec((K * (IFZ // 2), K * AHZ), lambda i: (0, 0)),
        ],
        out_specs=[
            pl.BlockSpec((B1, DT), lambda i: (i, 0)),
            pl.BlockSpec((B1, AHZ * AFZ), lambda i: (i, 0)),
            pl.BlockSpec((B1, 144), lambda i: (i, 0)),
            pl.BlockSpec((B1, K * AHZ), lambda i: (i, 0)),
        ],
        out_shape=[
            jax.ShapeDtypeStruct((N, DT), jnp.bfloat16),
            jax.ShapeDtypeStruct((N, AHZ * AFZ), jnp.float32),
            jax.ShapeDtypeStruct((N, 144), jnp.float32),
            jax.ShapeDtypeStruct((N, K * AHZ), jnp.float32),
        ],
        interpret=interpret,
    )(x1, x2f, pos_emb, affq, wcat, biaw2)


NW = 32
EPW = N * K // NW
CHUNK = 40
STEPS = EPW // CHUNK


def _gather_body(table_hbm, idx_hbm, out_hbm, idx_v, rows_v, sem):
    wid = lax.axis_index("s") * 2 + lax.axis_index("c")
    base = wid * EPW

    def step(s, carry):
        off = base + s * CHUNK
        pltpu.sync_copy(idx_hbm.at[pl.ds(off, CHUNK)], idx_v)
        pltpu.async_copy(table_hbm.at[idx_v], rows_v, sem).wait()
        pltpu.sync_copy(rows_v, out_hbm.at[pl.ds(off, CHUNK)])
        return carry

    lax.fori_loop(0, STEPS, step, 0)


def _stage2(table, edge_flat):
    mesh = plsc.VectorSubcoreMesh(core_axis_name="c", subcore_axis_name="s")
    k = functools.partial(
        pl.kernel,
        out_type=jax.ShapeDtypeStruct((N * K, DT), jnp.bfloat16),
        mesh=mesh,
        scratch_types=[
            pltpu.VMEM((CHUNK,), jnp.int32),
            pltpu.VMEM((CHUNK, DT), jnp.bfloat16),
            pltpu.SemaphoreType.DMA,
        ],
        compiler_params=pltpu.CompilerParams(use_tc_tiling_on_sc=False),
    )(_gather_body)
    return k(table, edge_flat)


def _s3_body(g_ref, q1r_ref, tq_ref, bias_ref, x2f_ref, aff_ref, x1_ref,
             gamma_ref, backw_ref, backb_ref, lng_ref, lnb_ref, out_ref):
    b = B3
    g = g_ref[:].reshape(b, K, DT)
    aff = aff_ref[:]

    k1r = g[:, :, 0:192].astype(jnp.float32)
    q1r = q1r_ref[:].reshape(b, 1, AHZ, AFZ)
    s1 = (k1r.reshape(b, K, AHZ, AFZ) * q1r).sum(axis=-1)

    k2 = g[:, :, 384:528].astype(jnp.float32)
    tks = []
    for i in range(3):
        acc = aff[:, 4 * i + 3].reshape(b, 1, 1)
        for j in range(3):
            acc = acc + aff[:, 4 * i + j].reshape(b, 1, 1) * k2[:, :, 48 * j:48 * (j + 1)]
        tks.append(acc)
    tq = tq_ref[:].reshape(b, 1, 144)
    d0 = tq[:, :, 0:48] - tks[0]
    d1 = tq[:, :, 48:96] - tks[1]
    d2 = tq[:, :, 96:144] - tks[2]
    dsq = d0 * d0 + d1 * d1 + d2 * d2
    s2 = -dsq.reshape(b, K, AHZ, QPZ).sum(axis=-1)

    gam = gamma_ref[:].reshape(1, 1, AHZ)
    scores = WL * (s1 * (1.0 / math.sqrt(AFZ)) + bias_ref[:].reshape(b, K, AHZ)
                   + (0.1 * WC) * gam * s2)
    m = scores.max(axis=1, keepdims=True)
    e = jnp.exp(scores - m)
    w = e / e.sum(axis=1, keepdims=True)

    v1 = g[:, :, 192:384].astype(jnp.float32)
    w16 = jnp.broadcast_to(w[:, :, :, None], (b, K, AHZ, AFZ)).reshape(b, K, 192)
    out1 = (w16 * v1).sum(axis=1)

    x2r = x2f_ref[:].reshape(b, K, IFZ // 2)
    out2 = lax.dot_general(w, x2r, (((1,), (1,)), ((0,), (0,))),
                           preferred_element_type=jnp.float32)
    out2f = out2.reshape(b, AHZ * (IFZ // 2))

    tv = g[:, :, 528:816].astype(jnp.float32)
    w8 = jnp.broadcast_to(w[:, :, :, None], (b, K, AHZ, VPZ)).reshape(b, K, 96)
    o = [(w8 * tv[:, :, 96 * c:96 * (c + 1)]).sum(axis=1) for c in range(3)]
    oc = [o[c] - aff[:, 4 * c + 3].reshape(b, 1) for c in range(3)]
    ipa = [oc[0] * aff[:, 0 + i].reshape(b, 1)
           + oc[1] * aff[:, 4 + i].reshape(b, 1)
           + oc[2] * aff[:, 8 + i].reshape(b, 1) for i in range(3)]
    norm = jnp.sqrt(ipa[0] * ipa[0] + ipa[1] * ipa[1] + ipa[2] * ipa[2] + 1e-12)

    cat = jnp.concatenate([out1, out2f, ipa[0], ipa[1], ipa[2], norm], axis=1)
    out = lax.dot_general(cat, backw_ref[:], (((1,), (1,)), ((), ())),
                          preferred_element_type=jnp.float32) + backb_ref[:]
    h = math.sqrt(2.0) * x1_ref[:] + out
    mu = jnp.mean(h, axis=-1, keepdims=True)
    var = jnp.mean(jnp.square(h - mu), axis=-1, keepdims=True)
    out_ref[:] = lng_ref[:] * (h - mu) / jnp.sqrt(var + 1e-5) + lnb_ref[:]


def _stage3(g, q1r, tq, bias, x2f, affq, x1, gamma2, backwp, backb2, lng2, lnb2,
            *, interpret=False):
    grid = (N // B3,)
    return pl.pallas_call(
        _s3_body,
        grid=grid,
        in_specs=[
            pl.BlockSpec((B3 * K, DT), lambda i: (i, 0)),
            pl.BlockSpec((B3, AHZ * AFZ), lambda i: (i, 0)),
            pl.BlockSpec((B3, 144), lambda i: (i, 0)),
            pl.BlockSpec((B3, K * AHZ), lambda i: (i, 0)),
            pl.BlockSpec((B3, K * (IFZ // 2)), lambda i: (i, 0)),
            pl.BlockSpec((B3, 16), lambda i: (i, 0)),
            pl.BlockSpec((B3, IFZ), lambda i: (i, 0)),
            pl.BlockSpec((1, AHZ), lambda i: (0, 0)),
            pl.BlockSpec((IFZ, 1344), lambda i: (0, 0)),
            pl.BlockSpec((1, IFZ), lambda i: (0, 0)),
            pl.BlockSpec((1, IFZ), lambda i: (0, 0)),
            pl.BlockSpec((1, IFZ), lambda i: (0, 0)),
        ],
        out_specs=pl.BlockSpec((B3, IFZ), lambda i: (i, 0)),
        out_shape=jax.ShapeDtypeStruct((N, IFZ), jnp.float32),
        interpret=interpret,
    )(g, q1r, tq, bias, x2f, affq, x1, gamma2, backwp, backb2, lng2, lnb2)


def kernel(x1, x2, affines, pos_emb, edge_index, q1_w, k1_w, v1_w, q2_w,
           k2_w, v2_w, bia_w, back_w, back_b, gamma, ln_g, ln_b):
    pr = _perm_rope()
    pq = _perm_cmajor(QPZ)
    pv = _perm_cmajor(VPZ)
    wcat = jnp.concatenate([q1_w[pr], k1_w[pr], v1_w, q2_w[pq], k2_w[pq],
                            v2_w[pv]], axis=0)
    backwp = back_w[:, _perm_backw_cols()]
    affq = affines.reshape(N, 16)
    x2f = x2.reshape(N, K * (IFZ // 2))
    biaw2 = jnp.zeros((K * (IFZ // 2), K * AHZ), jnp.float32)
    kk = np.arange(K)
    biaw2 = biaw2.at[kk[:, None, None] * (IFZ // 2) + np.arange(IFZ // 2)[None, None, :],
                     kk[:, None, None] * AHZ + np.arange(AHZ)[None, :, None]].set(bia_w[None, :, :])

    table, q1r, tq, bias = _stage1(x1, x2f, pos_emb, affq, wcat, biaw2)
    g = _stage2(table, edge_index.reshape(-1))
    return _stage3(g, q1r, tq, bias, x2f, affq, x1,
                   gamma.reshape(1, AHZ), backwp,
                   back_b.reshape(1, IFZ), ln_g.reshape(1, IFZ),
                   ln_b.reshape(1, IFZ))


# speedup vs baseline: 1.0257x; 1.0257x over previous
import functools
import math

import jax
import jax.numpy as jnp
import numpy as np
from jax import lax
from jax.experimental import pallas as pl
from jax.experimental.pallas import tpu as pltpu
from jax.experimental.pallas import tpu_sc as plsc

N = 10000
K = 16
IFZ = 128
AHZ = 12
AFZ = 16
QPZ = 4
VPZ = 8
DT = 832
DTU = 816

WL = math.sqrt(1.0 / 3.0)
WC = math.sqrt(2.0 / (9.0 * QPZ))

B1 = 400
B3 = 80


def _perm_rope():
    p = []
    for a in range(AHZ):
        p += [a * AFZ + i for i in range(0, AFZ, 2)]
        p += [a * AFZ + i for i in range(1, AFZ, 2)]
    return np.array(p)


def _perm_cmajor(npts):
    p = np.empty(3 * AHZ * npts, dtype=np.int64)
    for c in range(3):
        for a in range(AHZ):
            for q in range(npts):
                p[c * AHZ * npts + a * npts + q] = (a * npts + q) * 3 + c
    return p


def _perm_backw_cols():
    cols = np.arange(AHZ * AFZ + AHZ * (IFZ // 2) + AHZ * VPZ * 4)
    base = AHZ * AFZ + AHZ * (IFZ // 2)
    ipa = np.empty(3 * AHZ * VPZ, dtype=np.int64)
    for c in range(3):
        for av in range(AHZ * VPZ):
            ipa[c * AHZ * VPZ + av] = base + av * 3 + c
    cols[base:base + 3 * AHZ * VPZ] = ipa
    return cols


def _s1_body(x1_ref, x2f_ref, pe_ref, aff_ref, w_ref, biaw2_ref,
             table_ref, q1r_ref, tq_ref, bias_ref):
    x1 = x1_ref[:]
    proj = lax.dot_general(x1, w_ref[:], (((1,), (1,)), ((), ())),
                           preferred_element_type=jnp.float32)
    pe = pe_ref[:]
    cos = jnp.cos(pe)[:, None, :]
    sin = jnp.sin(pe)[:, None, :]
    b = x1.shape[0]

    def rope(q):
        q3 = q.reshape(b, AHZ, AFZ)
        v1 = q3[:, :, 0:8]
        v2 = q3[:, :, 8:16]
        r1 = v1 * cos - v2 * sin
        r2 = v1 * sin + v2 * cos
        return jnp.concatenate([r1, r2], axis=-1).reshape(b, AHZ * AFZ)

    aff = aff_ref[:]

    def affmul(v, s):
        outs = []
        for i in range(3):
            acc = aff[:, 4 * i + 3:4 * i + 4]
            for j in range(3):
                acc = acc + aff[:, 4 * i + j:4 * i + j + 1] * v[:, j * s:(j + 1) * s]
            outs.append(acc)
        return jnp.concatenate(outs, axis=-1)

    q1r_ref[:] = rope(proj[:, 0:192])
    k1r = rope(proj[:, 192:384])
    v1 = proj[:, 384:576]
    tq_ref[:] = affmul(proj[:, 576:720], 48)
    k2 = proj[:, 720:864]
    tv = affmul(proj[:, 864:1152], 96)
    pad = jnp.zeros((b, DT - DTU), jnp.float32)
    table_ref[:] = jnp.concatenate([k1r, v1, k2, tv, pad],
                                   axis=-1).astype(jnp.bfloat16)
    bias_ref[:] = lax.dot_general(x2f_ref[:], biaw2_ref[:],
                                  (((1,), (0,)), ((), ())),
                                  preferred_element_type=jnp.float32)


def _stage1(x1, x2f, pos_emb, affq, wcat, biaw2, *, interpret=False):
    grid = (N // B1,)
    return pl.pallas_call(
        _s1_body,
        grid=grid,
        in_specs=[
            pl.BlockSpec((B1, IFZ), lambda i: (i, 0)),
            pl.BlockSpec((B1, K * (IFZ // 2)), lambda i: (i, 0)),
            pl.BlockSpec((B1, AFZ // 2), lambda i: (i, 0)),
            pl.BlockSpec((B1, 16), lambda i: (i, 0)),
            pl.BlockSpec((1152, IFZ), lambda i: (0, 0)),
            pl.BlockSpec((K * (IFZ // 2), K * AHZ), lambda i: (0, 0)),
        ],
        out_specs=[
            pl.BlockSpec((B1, DT), lambda i: (i, 0)),
            pl.BlockSpec((B1, AHZ * AFZ), lambda i: (i, 0)),
            pl.BlockSpec((B1, 144), lambda i: (i, 0)),
            pl.BlockSpec((B1, K * AHZ), lambda i: (i, 0)),
        ],
        out_shape=[
            jax.ShapeDtypeStruct((N, DT), jnp.bfloat16),
            jax.ShapeDtypeStruct((N, AHZ * AFZ), jnp.float32),
            jax.ShapeDtypeStruct((N, 144), jnp.float32),
            jax.ShapeDtypeStruct((N, K * AHZ), jnp.float32),
        ],
        interpret=interpret,
    )(x1, x2f, pos_emb, affq, wcat, biaw2)


NW = 32
EPW = N * K // NW
CHUNK = 40
STEPS = EPW // CHUNK


NBUF = 5


def _gather_body(table_hbm, idx_hbm, out_hbm, idx_v, *bufs):
    rows = bufs[0:NBUF]
    gs = bufs[NBUF:2 * NBUF]
    os_ = bufs[2 * NBUF:3 * NBUF]
    wid = lax.axis_index("s") * 2 + lax.axis_index("c")
    base = wid * EPW
    pltpu.sync_copy(idx_hbm.at[pl.ds(base, EPW)], idx_v)
    for s in range(NBUF - 1):
        pltpu.async_copy(table_hbm.at[idx_v.at[pl.ds(s * CHUNK, CHUNK)]],
                         rows[s], gs[s])

    def outer(t, carry):
        for b in range(NBUF):
            s = NBUF * t + b
            nxt = s + NBUF - 1
            nb = (b + NBUF - 1) % NBUF

            @pl.when(jnp.logical_and(nxt < STEPS, s >= 1))
            def _():
                pltpu.make_async_copy(
                    rows[nb], out_hbm.at[pl.ds(base, CHUNK)], os_[nb]).wait()

            @pl.when(nxt < STEPS)
            def _():
                pltpu.async_copy(
                    table_hbm.at[idx_v.at[pl.ds(nxt * CHUNK, CHUNK)]],
                    rows[nb], gs[nb])

            pltpu.make_async_copy(
                table_hbm.at[pl.ds(0, CHUNK)], rows[b], gs[b]).wait()
            pltpu.async_copy(rows[b],
                             out_hbm.at[pl.ds(base + s * CHUNK, CHUNK)],
                             os_[b])
        return carry

    lax.fori_loop(0, STEPS // NBUF, outer, 0)
    for b in range(NBUF):
        pltpu.make_async_copy(rows[b], out_hbm.at[pl.ds(base, CHUNK)],
                              os_[b]).wait()


def _stage2(table, edge_flat):
    mesh = plsc.VectorSubcoreMesh(core_axis_name="c", subcore_axis_name="s")
    scratch = ([pltpu.VMEM((EPW,), jnp.int32)]
               + [pltpu.VMEM((CHUNK, DT), jnp.bfloat16) for _ in range(NBUF)]
               + [pltpu.SemaphoreType.DMA for _ in range(2 * NBUF)])
    k = functools.partial(
        pl.kernel,
        out_type=jax.ShapeDtypeStruct((N * K, DT), jnp.bfloat16),
        mesh=mesh,
        scratch_types=scratch,
        compiler_params=pltpu.CompilerParams(use_tc_tiling_on_sc=False),
    )(_gather_body)
    return k(table, edge_flat)


def _s3_body(g_ref, q1r_ref, tq_ref, bias_ref, x2f_ref, aff_ref, x1_ref,
             gamma_ref, backw_ref, backb_ref, lng_ref, lnb_ref, out_ref):
    b = B3
    g = g_ref[:].reshape(b, K, DT)
    aff = aff_ref[:]

    k1r = g[:, :, 0:192].astype(jnp.float32)
    q1r = q1r_ref[:].reshape(b, 1, AHZ, AFZ)
    s1 = (k1r.reshape(b, K, AHZ, AFZ) * q1r).sum(axis=-1)

    k2 = g[:, :, 384:528].astype(jnp.float32)
    tks = []
    for i in range(3):
        acc = aff[:, 4 * i + 3].reshape(b, 1, 1)
        for j in range(3):
            acc = acc + aff[:, 4 * i + j].reshape(b, 1, 1) * k2[:, :, 48 * j:48 * (j + 1)]
        tks.append(acc)
    tq = tq_ref[:].reshape(b, 1, 144)
    d0 = tq[:, :, 0:48] - tks[0]
    d1 = tq[:, :, 48:96] - tks[1]
    d2 = tq[:, :, 96:144] - tks[2]
    dsq = d0 * d0 + d1 * d1 + d2 * d2
    s2 = -dsq.reshape(b, K, AHZ, QPZ).sum(axis=-1)

    gam = gamma_ref[:].reshape(1, 1, AHZ)
    scores = WL * (s1 * (1.0 / math.sqrt(AFZ)) + bias_ref[:].reshape(b, K, AHZ)
                   + (0.1 * WC) * gam * s2)
    m = scores.max(axis=1, keepdims=True)
    e = jnp.exp(scores - m)
    w = e / e.sum(axis=1, keepdims=True)

    v1 = g[:, :, 192:384].astype(jnp.float32)
    w16 = jnp.broadcast_to(w[:, :, :, None], (b, K, AHZ, AFZ)).reshape(b, K, 192)
    out1 = (w16 * v1).sum(axis=1)

    x2r = x2f_ref[:].reshape(b, K, IFZ // 2)
    out2 = lax.dot_general(w, x2r, (((1,), (1,)), ((0,), (0,))),
                           preferred_element_type=jnp.float32)
    out2f = out2.reshape(b, AHZ * (IFZ // 2))

    tv = g[:, :, 528:816].astype(jnp.float32)
    w8 = jnp.broadcast_to(w[:, :, :, None], (b, K, AHZ, VPZ)).reshape(b, K, 96)
    o = [(w8 * tv[:, :, 96 * c:96 * (c + 1)]).sum(axis=1) for c in range(3)]
    oc = [o[c] - aff[:, 4 * c + 3].reshape(b, 1) for c in range(3)]
    ipa = [oc[0] * aff[:, 0 + i].reshape(b, 1)
           + oc[1] * aff[:, 4 + i].reshape(b, 1)
           + oc[2] * aff[:, 8 + i].reshape(b, 1) for i in range(3)]
    norm = jnp.sqrt(ipa[0] * ipa[0] + ipa[1] * ipa[1] + ipa[2] * ipa[2] + 1e-12)

    cat = jnp.concatenate([out1, out2f, ipa[0], ipa[1], ipa[2], norm], axis=1)
    out = lax.dot_general(cat, backw_ref[:], (((1,), (1,)), ((), ())),
                          preferred_element_type=jnp.float32) + backb_ref[:]
    h = math.sqrt(2.0) * x1_ref[:] + out
    mu = jnp.mean(h, axis=-1, keepdims=True)
    var = jnp.mean(jnp.square(h - mu), axis=-1, keepdims=True)
    out_ref[:] = lng_ref[:] * (h - mu) / jnp.sqrt(var + 1e-5) + lnb_ref[:]


def _stage3(g, q1r, tq, bias, x2f, affq, x1, gamma2, backwp, backb2, lng2, lnb2,
            *, interpret=False):
    grid = (N // B3,)
    return pl.pallas_call(
        _s3_body,
        grid=grid,
        in_specs=[
            pl.BlockSpec((B3 * K, DT), lambda i: (i, 0)),
            pl.BlockSpec((B3, AHZ * AFZ), lambda i: (i, 0)),
            pl.BlockSpec((B3, 144), lambda i: (i, 0)),
            pl.BlockSpec((B3, K * AHZ), lambda i: (i, 0)),
            pl.BlockSpec((B3, K * (IFZ // 2)), lambda i: (i, 0)),
            pl.BlockSpec((B3, 16), lambda i: (i, 0)),
            pl.BlockSpec((B3, IFZ), lambda i: (i, 0)),
            pl.BlockSpec((1, AHZ), lambda i: (0, 0)),
            pl.BlockSpec((IFZ, 1344), lambda i: (0, 0)),
            pl.BlockSpec((1, IFZ), lambda i: (0, 0)),
            pl.BlockSpec((1, IFZ), lambda i: (0, 0)),
            pl.BlockSpec((1, IFZ), lambda i: (0, 0)),
        ],
        out_specs=pl.BlockSpec((B3, IFZ), lambda i: (i, 0)),
        out_shape=jax.ShapeDtypeStruct((N, IFZ), jnp.float32),
        interpret=interpret,
    )(g, q1r, tq, bias, x2f, affq, x1, gamma2, backwp, backb2, lng2, lnb2)


def kernel(x1, x2, affines, pos_emb, edge_index, q1_w, k1_w, v1_w, q2_w,
           k2_w, v2_w, bia_w, back_w, back_b, gamma, ln_g, ln_b):
    pr = _perm_rope()
    pq = _perm_cmajor(QPZ)
    pv = _perm_cmajor(VPZ)
    wcat = jnp.concatenate([q1_w[pr], k1_w[pr], v1_w, q2_w[pq], k2_w[pq],
                            v2_w[pv]], axis=0)
    backwp = back_w[:, _perm_backw_cols()]
    affq = affines.reshape(N, 16)
    x2f = x2.reshape(N, K * (IFZ // 2))
    biaw2 = jnp.zeros((K * (IFZ // 2), K * AHZ), jnp.float32)
    kk = np.arange(K)
    biaw2 = biaw2.at[kk[:, None, None] * (IFZ // 2) + np.arange(IFZ // 2)[None, None, :],
                     kk[:, None, None] * AHZ + np.arange(AHZ)[None, :, None]].set(bia_w[None, :, :])

    table, q1r, tq, bias = _stage1(x1, x2f, pos_emb, affq, wcat, biaw2)
    g = _stage2(table, edge_index.reshape(-1))
    return _stage3(g, q1r, tq, bias, x2f, affq, x1,
                   gamma.reshape(1, AHZ), backwp,
                   back_b.reshape(1, IFZ), ln_g.reshape(1, IFZ),
                   ln_b.reshape(1, IFZ))


# speedup vs baseline: 2.2925x; 2.2350x over previous
import functools
import math

import jax
import jax.numpy as jnp
import numpy as np
from jax import lax
from jax.experimental import pallas as pl
from jax.experimental.pallas import tpu as pltpu
from jax.experimental.pallas import tpu_sc as plsc

N = 10000
K = 16
IFZ = 128
AHZ = 12
AFZ = 16
QPZ = 4
VPZ = 8
DT = 832
DTU = 816

WL = math.sqrt(1.0 / 3.0)
WC = math.sqrt(2.0 / (9.0 * QPZ))

B1 = 400
B3 = 80


def _perm_rope():
    p = []
    for a in range(AHZ):
        p += [a * AFZ + i for i in range(0, AFZ, 2)]
    for a in range(AHZ):
        p += [a * AFZ + i for i in range(1, AFZ, 2)]
    return np.array(p)


def _perm_cmajor(npts):
    p = np.empty(3 * AHZ * npts, dtype=np.int64)
    for c in range(3):
        for a in range(AHZ):
            for q in range(npts):
                p[c * AHZ * npts + a * npts + q] = (a * npts + q) * 3 + c
    return p


def _perm_backw_cols():
    cols = np.arange(AHZ * AFZ + AHZ * (IFZ // 2) + AHZ * VPZ * 4)
    base = AHZ * AFZ + AHZ * (IFZ // 2)
    ipa = np.empty(3 * AHZ * VPZ, dtype=np.int64)
    for c in range(3):
        for av in range(AHZ * VPZ):
            ipa[c * AHZ * VPZ + av] = base + av * 3 + c
    cols[base:base + 3 * AHZ * VPZ] = ipa
    return cols


def _s1_body(x1_ref, x2f_ref, pe_ref, aff_ref, w_ref, biaw2_ref,
             table_ref, q1r_ref, tq_ref, bias_ref):
    x1 = x1_ref[:]
    proj = lax.dot_general(x1, w_ref[:], (((1,), (1,)), ((), ())),
                           preferred_element_type=jnp.float32)
    pe = pe_ref[:]
    cos = jnp.cos(pe)
    sin = jnp.sin(pe)
    b = x1.shape[0]

    def rope(q):
        v1 = q[:, 0:96]
        v2 = q[:, 96:192]
        return jnp.concatenate([v1 * cos - v2 * sin, v1 * sin + v2 * cos],
                               axis=-1)

    aff = aff_ref[:]

    def affmul(v, s):
        outs = []
        for i in range(3):
            acc = aff[:, 4 * i + 3:4 * i + 4]
            for j in range(3):
                acc = acc + aff[:, 4 * i + j:4 * i + j + 1] * v[:, j * s:(j + 1) * s]
            outs.append(acc)
        return jnp.concatenate(outs, axis=-1)

    q1r_ref[:] = rope(proj[:, 0:192])
    k1r = rope(proj[:, 192:384])
    v1 = proj[:, 384:576]
    tq_ref[:] = affmul(proj[:, 576:720], 48)
    k2 = proj[:, 720:864]
    tv = affmul(proj[:, 864:1152], 96)
    pad = jnp.zeros((b, DT - DTU), jnp.float32)
    table_ref[:] = jnp.concatenate([k1r, v1, k2, tv, pad],
                                   axis=-1).astype(jnp.bfloat16)
    bias_ref[:] = lax.dot_general(x2f_ref[:], biaw2_ref[:],
                                  (((1,), (0,)), ((), ())),
                                  preferred_element_type=jnp.float32)


def _stage1(x1, x2f, pos_emb, affq, wcat, biaw2, *, interpret=False):
    grid = (N // B1,)
    return pl.pallas_call(
        _s1_body,
        grid=grid,
        in_specs=[
            pl.BlockSpec((B1, IFZ), lambda i: (i, 0)),
            pl.BlockSpec((B1, K * (IFZ // 2)), lambda i: (i, 0)),
            pl.BlockSpec((B1, 96), lambda i: (i, 0)),
            pl.BlockSpec((B1, 16), lambda i: (i, 0)),
            pl.BlockSpec((1152, IFZ), lambda i: (0, 0)),
            pl.BlockSpec((K * (IFZ // 2), K * AHZ), lambda i: (0, 0)),
        ],
        out_specs=[
            pl.BlockSpec((B1, DT), lambda i: (i, 0)),
            pl.BlockSpec((B1, AHZ * AFZ), lambda i: (i, 0)),
            pl.BlockSpec((B1, 144), lambda i: (i, 0)),
            pl.BlockSpec((B1, K * AHZ), lambda i: (i, 0)),
        ],
        out_shape=[
            jax.ShapeDtypeStruct((N, DT), jnp.bfloat16),
            jax.ShapeDtypeStruct((N, AHZ * AFZ), jnp.float32),
            jax.ShapeDtypeStruct((N, 144), jnp.float32),
            jax.ShapeDtypeStruct((N, K * AHZ), jnp.float32),
        ],
        interpret=interpret,
    )(x1, x2f, pos_emb, affq, wcat, biaw2)


NW = 32
EPW = N * K // NW
CHUNK = 40
STEPS = EPW // CHUNK


NBUF = 5


def _gather_body(table_hbm, idx_hbm, out_hbm, idx_v, *bufs):
    rows = bufs[0:NBUF]
    gs = bufs[NBUF:2 * NBUF]
    os_ = bufs[2 * NBUF:3 * NBUF]
    wid = lax.axis_index("s") * 2 + lax.axis_index("c")
    base = wid * EPW
    pltpu.sync_copy(idx_hbm.at[pl.ds(base, EPW)], idx_v)
    for s in range(NBUF - 1):
        pltpu.async_copy(table_hbm.at[idx_v.at[pl.ds(s * CHUNK, CHUNK)]],
                         rows[s], gs[s])

    def outer(t, carry):
        for b in range(NBUF):
            s = NBUF * t + b
            nxt = s + NBUF - 1
            nb = (b + NBUF - 1) % NBUF

            @pl.when(jnp.logical_and(nxt < STEPS, s >= 1))
            def _():
                pltpu.make_async_copy(
                    rows[nb], out_hbm.at[pl.ds(base, CHUNK)], os_[nb]).wait()

            @pl.when(nxt < STEPS)
            def _():
                pltpu.async_copy(
                    table_hbm.at[idx_v.at[pl.ds(nxt * CHUNK, CHUNK)]],
                    rows[nb], gs[nb])

            pltpu.make_async_copy(
                table_hbm.at[pl.ds(0, CHUNK)], rows[b], gs[b]).wait()
            pltpu.async_copy(rows[b],
                             out_hbm.at[pl.ds(base + s * CHUNK, CHUNK)],
                             os_[b])
        return carry

    lax.fori_loop(0, STEPS // NBUF, outer, 0)
    for b in range(NBUF):
        pltpu.make_async_copy(rows[b], out_hbm.at[pl.ds(base, CHUNK)],
                              os_[b]).wait()


def _stage2(table, edge_flat):
    mesh = plsc.VectorSubcoreMesh(core_axis_name="c", subcore_axis_name="s")
    scratch = ([pltpu.VMEM((EPW,), jnp.int32)]
               + [pltpu.VMEM((CHUNK, DT), jnp.bfloat16) for _ in range(NBUF)]
               + [pltpu.SemaphoreType.DMA for _ in range(2 * NBUF)])
    k = functools.partial(
        pl.kernel,
        out_type=jax.ShapeDtypeStruct((N * K, DT), jnp.bfloat16),
        mesh=mesh,
        scratch_types=scratch,
        compiler_params=pltpu.CompilerParams(use_tc_tiling_on_sc=False),
    )(_gather_body)
    return k(table, edge_flat)


def _s3_body(g_ref, q1r_ref, tq_ref, bias_ref, x2f_ref, aff_ref, x1_ref,
             gamma_ref, backw_ref, backb_ref, lng_ref, lnb_ref, out_ref):
    b = B3
    g = g_ref[:].reshape(b, K, DT)
    aff = aff_ref[:]

    def seg(npts):
        r = lax.broadcasted_iota(jnp.int32, (AHZ * npts, AHZ), 0) // npts
        c = lax.broadcasted_iota(jnp.int32, (AHZ * npts, AHZ), 1)
        return (r == c).astype(jnp.float32)

    s16 = seg(AFZ)
    s4 = seg(QPZ)
    s8 = seg(VPZ)

    rr = lax.broadcasted_iota(jnp.int32, (192, AHZ), 0)
    cc = lax.broadcasted_iota(jnp.int32, (192, AHZ), 1)
    srope = ((rr % 96) // 8 == cc).astype(jnp.float32)
    k1r = g[:, :, 0:192].astype(jnp.float32)
    q1r = q1r_ref[:].reshape(b, 1, 192)
    p1 = k1r * q1r
    s1 = lax.dot_general(p1, srope, (((2,), (0,)), ((), ())),
                         preferred_element_type=jnp.float32)

    k2 = g[:, :, 384:528].astype(jnp.float32)
    tks = []
    for i in range(3):
        acc = aff[:, 4 * i + 3].reshape(b, 1, 1)
        for j in range(3):
            acc = acc + aff[:, 4 * i + j].reshape(b, 1, 1) * k2[:, :, 48 * j:48 * (j + 1)]
        tks.append(acc)
    tq = tq_ref[:].reshape(b, 1, 144)
    d0 = tq[:, :, 0:48] - tks[0]
    d1 = tq[:, :, 48:96] - tks[1]
    d2 = tq[:, :, 96:144] - tks[2]
    dsq = d0 * d0 + d1 * d1 + d2 * d2
    s2 = -lax.dot_general(dsq, s4, (((2,), (0,)), ((), ())),
                          preferred_element_type=jnp.float32)

    gam = gamma_ref[:].reshape(1, 1, AHZ)
    scores = WL * (s1 * (1.0 / math.sqrt(AFZ)) + bias_ref[:].reshape(b, K, AHZ)
                   + (0.1 * WC) * gam * s2)
    m = scores.max(axis=1, keepdims=True)
    e = jnp.exp(scores - m)
    w = e / e.sum(axis=1, keepdims=True)

    v1 = g[:, :, 192:384].astype(jnp.float32)
    w16 = lax.dot_general(w, s16, (((2,), (1,)), ((), ())),
                          preferred_element_type=jnp.float32)
    out1 = (w16 * v1).sum(axis=1)

    x2r = x2f_ref[:].reshape(b, K, IFZ // 2)
    out2 = lax.dot_general(w, x2r, (((1,), (1,)), ((0,), (0,))),
                           preferred_element_type=jnp.float32)
    out2f = out2.reshape(b, AHZ * (IFZ // 2))

    tv = g[:, :, 528:816].astype(jnp.float32)
    w8 = lax.dot_general(w, s8, (((2,), (1,)), ((), ())),
                         preferred_element_type=jnp.float32)
    o = [(w8 * tv[:, :, 96 * c:96 * (c + 1)]).sum(axis=1) for c in range(3)]
    oc = [o[c] - aff[:, 4 * c + 3].reshape(b, 1) for c in range(3)]
    ipa = [oc[0] * aff[:, 0 + i].reshape(b, 1)
           + oc[1] * aff[:, 4 + i].reshape(b, 1)
           + oc[2] * aff[:, 8 + i].reshape(b, 1) for i in range(3)]
    norm = jnp.sqrt(ipa[0] * ipa[0] + ipa[1] * ipa[1] + ipa[2] * ipa[2] + 1e-12)

    cat = jnp.concatenate([out1, out2f, ipa[0], ipa[1], ipa[2], norm], axis=1)
    out = lax.dot_general(cat, backw_ref[:], (((1,), (1,)), ((), ())),
                          preferred_element_type=jnp.float32) + backb_ref[:]
    h = math.sqrt(2.0) * x1_ref[:] + out
    mu = jnp.mean(h, axis=-1, keepdims=True)
    var = jnp.mean(jnp.square(h - mu), axis=-1, keepdims=True)
    out_ref[:] = lng_ref[:] * (h - mu) / jnp.sqrt(var + 1e-5) + lnb_ref[:]


def _stage3(g, q1r, tq, bias, x2f, affq, x1, gamma2, backwp, backb2, lng2, lnb2,
            *, interpret=False):
    grid = (N // B3,)
    return pl.pallas_call(
        _s3_body,
        grid=grid,
        in_specs=[
            pl.BlockSpec((B3 * K, DT), lambda i: (i, 0)),
            pl.BlockSpec((B3, AHZ * AFZ), lambda i: (i, 0)),
            pl.BlockSpec((B3, 144), lambda i: (i, 0)),
            pl.BlockSpec((B3, K * AHZ), lambda i: (i, 0)),
            pl.BlockSpec((B3, K * (IFZ // 2)), lambda i: (i, 0)),
            pl.BlockSpec((B3, 16), lambda i: (i, 0)),
            pl.BlockSpec((B3, IFZ), lambda i: (i, 0)),
            pl.BlockSpec((1, AHZ), lambda i: (0, 0)),
            pl.BlockSpec((IFZ, 1344), lambda i: (0, 0)),
            pl.BlockSpec((1, IFZ), lambda i: (0, 0)),
            pl.BlockSpec((1, IFZ), lambda i: (0, 0)),
            pl.BlockSpec((1, IFZ), lambda i: (0, 0)),
        ],
        out_specs=pl.BlockSpec((B3, IFZ), lambda i: (i, 0)),
        out_shape=jax.ShapeDtypeStruct((N, IFZ), jnp.float32),
        interpret=interpret,
    )(g, q1r, tq, bias, x2f, affq, x1, gamma2, backwp, backb2, lng2, lnb2)


def kernel(x1, x2, affines, pos_emb, edge_index, q1_w, k1_w, v1_w, q2_w,
           k2_w, v2_w, bia_w, back_w, back_b, gamma, ln_g, ln_b):
    pr = _perm_rope()
    pq = _perm_cmajor(QPZ)
    pv = _perm_cmajor(VPZ)
    wcat = jnp.concatenate([q1_w[pr], k1_w[pr], v1_w, q2_w[pq], k2_w[pq],
                            v2_w[pv]], axis=0)
    backwp = back_w[:, _perm_backw_cols()]
    affq = affines.reshape(N, 16)
    x2f = x2.reshape(N, K * (IFZ // 2))
    biaw2 = jnp.zeros((K * (IFZ // 2), K * AHZ), jnp.float32)
    kk = np.arange(K)
    biaw2 = biaw2.at[kk[:, None, None] * (IFZ // 2) + np.arange(IFZ // 2)[None, None, :],
                     kk[:, None, None] * AHZ + np.arange(AHZ)[None, :, None]].set(bia_w[None, :, :])

    pe96 = jnp.tile(pos_emb, (1, AHZ))
    table, q1r, tq, bias = _stage1(x1, x2f, pe96, affq, wcat, biaw2)
    g = _stage2(table, edge_index.reshape(-1))
    return _stage3(g, q1r, tq, bias, x2f, affq, x1,
                   gamma.reshape(1, AHZ), backwp,
                   back_b.reshape(1, IFZ), ln_g.reshape(1, IFZ),
                   ln_b.reshape(1, IFZ))
